# baseline (speedup 1.0000x reference)
import functools

import jax
import jax.numpy as jnp
from jax import lax
from jax.experimental import pallas as pl
from jax.experimental.pallas import tpu as pltpu
from jax.experimental.pallas import tpu_sc as plsc

B, H, W, C = 4, 112, 112, 96
HD, WD = 2 * H, 2 * W
LANES = 16
CV = C // LANES
NC, NS = 2, 16
NW = NC * NS
UNITS_PER_IMG = H
UNITS = B * UNITS_PER_IMG
UPW = UNITS // NW

THIRD = 1.0 / 3.0
TWO_THIRD = 2.0 / 3.0


def _splat(val):
    return jnp.full((LANES,), val, jnp.float32)


def _col_mix(m_v, o_v):
    def jbody(j, carry):
        for cc in range(CV):
            s = pl.ds(cc * LANES, LANES)
            mj = m_v[j, s]
            mj1 = m_v[j + 1, s]
            t = 0.25 * (mj1 - mj)
            o_v[2 * j + 1, s] = mj + t
            o_v[2 * j + 2, s] = mj1 - t
        return carry

    lax.fori_loop(1, W - 2, jbody, 0)

    for cc in range(CV):
        s = pl.ds(cc * LANES, LANES)
        m0 = m_v[0, s]
        m1 = m_v[1, s]
        o_v[0, s] = THIRD * m0
        o_v[1, s] = m0
        o_v[2, s] = THIRD * m0 + TWO_THIRD * m1
        mt0 = m_v[W - 2, s]
        mt1 = m_v[W - 1, s]
        o_v[WD - 3, s] = TWO_THIRD * mt0 + THIRD * mt1
        o_v[WD - 2, s] = mt1
        o_v[WD - 1, s] = THIRD * mt1


def _body(x_hbm, out_hbm, a0_v, b0_v, a1_v, b1_v, o1_v, o2_v,
          in_sem0, in_sem1, out_sem1, out_sem2):
    wid = lax.axis_index("s") * NC + lax.axis_index("c")

    def unit_rows(t):
        g = wid * UPW + t
        bb = g // UNITS_PER_IMG
        p = g - bb * UNITS_PER_IMG
        is_edge = p == UNITS_PER_IMG - 1
        ra = jnp.where(is_edge, 0, jnp.minimum(p, H - 2))
        rb = jnp.where(is_edge, H - 1, ra + 1)
        return bb, p, is_edge, ra, rb

    def start_in(t, av, bv, sem):
        bb, _, _, ra, rb = unit_rows(t)
        pltpu.async_copy(x_hbm.at[bb, ra], av, sem)
        pltpu.async_copy(x_hbm.at[bb, rb], bv, sem)

    def wait_in(av, bv, sem):
        pltpu.make_async_copy(x_hbm.at[0, 0], av, sem).wait()
        pltpu.make_async_copy(x_hbm.at[0, 0], bv, sem).wait()

    def do_unit(t, av, bv, sem_cur, av_n, bv_n, sem_next):
        bb, p, is_edge, _, _ = unit_rows(t)
        u1 = jnp.where(is_edge, 0, 2 * p + 1)
        u2 = jnp.where(is_edge, HD - 1, 2 * p + 2)

        is_lo = p == 0
        is_hi = p == UNITS_PER_IMG - 2

        def wsel(lo, hi, edge, default):
            r = jnp.where(is_lo, _splat(lo), _splat(default))
            r = jnp.where(is_hi, _splat(hi), r)
            return jnp.where(is_edge, _splat(edge), r)

        a1 = wsel(1.0, TWO_THIRD, THIRD, 0.75)
        b1 = wsel(0.0, THIRD, 0.0, 0.25)
        a2 = wsel(THIRD, 0.0, 0.0, 0.25)
        b2 = wsel(TWO_THIRD, 1.0, THIRD, 0.75)

        wait_in(av, bv, sem_cur)

        @pl.when(t + 1 < UPW)
        def _():
            start_in(t + 1, av_n, bv_n, sem_next)

        def hrow(w, c2):
            for cc in range(CV):
                s = pl.ds(cc * LANES, LANES)
                xa = av[w, s]
                xb = bv[w, s]
                av[w, s] = a1 * xa + b1 * xb
                bv[w, s] = a2 * xa + b2 * xb
            return c2

        lax.fori_loop(0, W, hrow, 0)

        @pl.when(t > 0)
        def _():
            pltpu.make_async_copy(out_hbm.at[0, 0], o1_v, out_sem1).wait()

        _col_mix(av, o1_v)
        pltpu.async_copy(o1_v, out_hbm.at[bb, u1], out_sem1)

        @pl.when(t > 0)
        def _():
            pltpu.make_async_copy(out_hbm.at[0, 0], o2_v, out_sem2).wait()

        _col_mix(bv, o2_v)
        pltpu.async_copy(o2_v, out_hbm.at[bb, u2], out_sem2)

    start_in(0, a0_v, b0_v, in_sem0)

    def pair(q, carry):
        do_unit(2 * q, a0_v, b0_v, in_sem0, a1_v, b1_v, in_sem1)
        do_unit(2 * q + 1, a1_v, b1_v, in_sem1, a0_v, b0_v, in_sem0)
        return carry

    lax.fori_loop(0, UPW // 2, pair, 0)

    pltpu.make_async_copy(out_hbm.at[0, 0], o1_v, out_sem1).wait()
    pltpu.make_async_copy(out_hbm.at[0, 0], o2_v, out_sem2).wait()


@jax.jit
def _upsample(x):
    mesh = plsc.VectorSubcoreMesh(core_axis_name="c", subcore_axis_name="s")
    f = functools.partial(
        pl.kernel,
        mesh=mesh,
        out_type=jax.ShapeDtypeStruct((B, HD, WD, C), jnp.float32),
        scratch_types=[
            pltpu.VMEM((W, C), jnp.float32),
            pltpu.VMEM((W, C), jnp.float32),
            pltpu.VMEM((W, C), jnp.float32),
            pltpu.VMEM((W, C), jnp.float32),
            pltpu.VMEM((WD, C), jnp.float32),
            pltpu.VMEM((WD, C), jnp.float32),
            pltpu.SemaphoreType.DMA,
            pltpu.SemaphoreType.DMA,
            pltpu.SemaphoreType.DMA,
            pltpu.SemaphoreType.DMA,
        ],
    )(_body)
    return f(x)


def kernel(inputs):
    return _upsample(inputs)


# speedup vs baseline: 1.0717x; 1.0717x over previous
import functools

import jax
import jax.numpy as jnp
from jax import lax
from jax.experimental import pallas as pl
from jax.experimental.pallas import tpu as pltpu
from jax.experimental.pallas import tpu_sc as plsc

B, H, W, C = 4, 112, 112, 96
HD, WD = 2 * H, 2 * W
LANES = 16
CV = C // LANES
NC, NS = 2, 16
NW = NC * NS
UNITS_PER_IMG = H
UNITS = B * UNITS_PER_IMG
UPW = UNITS // NW

THIRD = 1.0 / 3.0
TWO_THIRD = 2.0 / 3.0


def _splat(val):
    return jnp.full((LANES,), val, jnp.float32)


def _col_mix(m_v, o_v):
    def jbody(j, carry):
        for cc in range(CV):
            s = pl.ds(cc * LANES, LANES)
            mj = m_v[j, s]
            mj1 = m_v[j + 1, s]
            o_v[2 * j + 1, s] = 0.75 * mj + 0.25 * mj1
            o_v[2 * j + 2, s] = 0.25 * mj + 0.75 * mj1
        return carry

    lax.fori_loop(1, W - 2, jbody, 0)

    for cc in range(CV):
        s = pl.ds(cc * LANES, LANES)
        m0 = m_v[0, s]
        m1 = m_v[1, s]
        o_v[0, s] = THIRD * m0
        o_v[1, s] = m0
        o_v[2, s] = THIRD * m0 + TWO_THIRD * m1
        mt0 = m_v[W - 2, s]
        mt1 = m_v[W - 1, s]
        o_v[WD - 3, s] = TWO_THIRD * mt0 + THIRD * mt1
        o_v[WD - 2, s] = mt1
        o_v[WD - 1, s] = THIRD * mt1


def _body(x_hbm, out_hbm, a0_v, b0_v, a1_v, b1_v, o1_v, o2_v,
          in_sem0, in_sem1, out_sem1, out_sem2):
    wid = lax.axis_index("s") * NC + lax.axis_index("c")

    def unit_rows(t):
        g = wid * UPW + t
        bb = g // UNITS_PER_IMG
        p = g - bb * UNITS_PER_IMG
        is_edge = p == UNITS_PER_IMG - 1
        ra = jnp.where(is_edge, 0, jnp.minimum(p, H - 2))
        rb = jnp.where(is_edge, H - 1, ra + 1)
        return bb, p, is_edge, ra, rb

    def start_in(t, av, bv, sem):
        bb, _, _, ra, rb = unit_rows(t)
        pltpu.async_copy(x_hbm.at[bb, ra], av, sem)
        pltpu.async_copy(x_hbm.at[bb, rb], bv, sem)

    def wait_in(av, bv, sem):
        pltpu.make_async_copy(x_hbm.at[0, 0], av, sem).wait()
        pltpu.make_async_copy(x_hbm.at[0, 0], bv, sem).wait()

    def do_unit(t, av, bv, sem_cur, av_n, bv_n, sem_next):
        bb, p, is_edge, _, _ = unit_rows(t)
        u1 = jnp.where(is_edge, 0, 2 * p + 1)
        u2 = jnp.where(is_edge, HD - 1, 2 * p + 2)

        is_lo = p == 0
        is_hi = p == UNITS_PER_IMG - 2

        def wsel(lo, hi, edge, default):
            r = jnp.where(is_lo, _splat(lo), _splat(default))
            r = jnp.where(is_hi, _splat(hi), r)
            return jnp.where(is_edge, _splat(edge), r)

        a1 = wsel(1.0, TWO_THIRD, THIRD, 0.75)
        b1 = wsel(0.0, THIRD, 0.0, 0.25)
        a2 = wsel(THIRD, 0.0, 0.0, 0.25)
        b2 = wsel(TWO_THIRD, 1.0, THIRD, 0.75)

        wait_in(av, bv, sem_cur)

        @pl.when(t + 1 < UPW)
        def _():
            start_in(t + 1, av_n, bv_n, sem_next)

        def hrow(w, c2):
            for cc in range(CV):
                s = pl.ds(cc * LANES, LANES)
                xa = av[w, s]
                xb = bv[w, s]
                av[w, s] = a1 * xa + b1 * xb
                bv[w, s] = a2 * xa + b2 * xb
            return c2

        lax.fori_loop(0, W, hrow, 0)

        @pl.when(t > 0)
        def _():
            pltpu.make_async_copy(out_hbm.at[0, 0], o1_v, out_sem1).wait()

        _col_mix(av, o1_v)
        pltpu.async_copy(o1_v, out_hbm.at[bb, u1], out_sem1)

        @pl.when(t > 0)
        def _():
            pltpu.make_async_copy(out_hbm.at[0, 0], o2_v, out_sem2).wait()

        _col_mix(bv, o2_v)
        pltpu.async_copy(o2_v, out_hbm.at[bb, u2], out_sem2)

    start_in(0, a0_v, b0_v, in_sem0)

    def pair(q, carry):
        do_unit(2 * q, a0_v, b0_v, in_sem0, a1_v, b1_v, in_sem1)
        do_unit(2 * q + 1, a1_v, b1_v, in_sem1, a0_v, b0_v, in_sem0)
        return carry

    lax.fori_loop(0, UPW // 2, pair, 0)

    pltpu.make_async_copy(out_hbm.at[0, 0], o1_v, out_sem1).wait()
    pltpu.make_async_copy(out_hbm.at[0, 0], o2_v, out_sem2).wait()


@jax.jit
def _upsample(x):
    mesh = plsc.VectorSubcoreMesh(core_axis_name="c", subcore_axis_name="s")
    f = functools.partial(
        pl.kernel,
        mesh=mesh,
        out_type=jax.ShapeDtypeStruct((B, HD, WD, C), jnp.float32),
        scratch_types=[
            pltpu.VMEM((W, C), jnp.float32),
            pltpu.VMEM((W, C), jnp.float32),
            pltpu.VMEM((W, C), jnp.float32),
            pltpu.VMEM((W, C), jnp.float32),
            pltpu.VMEM((WD, C), jnp.float32),
            pltpu.VMEM((WD, C), jnp.float32),
            pltpu.SemaphoreType.DMA,
            pltpu.SemaphoreType.DMA,
            pltpu.SemaphoreType.DMA,
            pltpu.SemaphoreType.DMA,
        ],
    )(_body)
    return f(x)


def kernel(inputs):
    return _upsample(inputs)


# speedup vs baseline: 1.5876x; 1.4814x over previous
import functools

import jax
import jax.numpy as jnp
from jax import lax
from jax.experimental import pallas as pl
from jax.experimental.pallas import tpu as pltpu
from jax.experimental.pallas import tpu_sc as plsc

B, H, W, C = 4, 112, 112, 96
HD, WD = 2 * H, 2 * W
LANES = 16
CV = C // LANES
NC, NS = 2, 16
NW = NC * NS
UNITS_PER_IMG = H
UNITS = B * UNITS_PER_IMG
UPW = UNITS // NW

THIRD = 1.0 / 3.0
TWO_THIRD = 2.0 / 3.0


def _splat(val):
    return jnp.full((LANES,), val, jnp.float32)


def _col_mix(m_v, o_v):
    @plsc.parallel_loop(1, W - 2)
    def jbody(j):
        for cc in range(CV):
            s = pl.ds(cc * LANES, LANES)
            mj = m_v[j, s]
            mj1 = m_v[j + 1, s]
            o_v[2 * j + 1, s] = 0.75 * mj + 0.25 * mj1
            o_v[2 * j + 2, s] = 0.25 * mj + 0.75 * mj1

    for cc in range(CV):
        s = pl.ds(cc * LANES, LANES)
        m0 = m_v[0, s]
        m1 = m_v[1, s]
        o_v[0, s] = THIRD * m0
        o_v[1, s] = m0
        o_v[2, s] = THIRD * m0 + TWO_THIRD * m1
        mt0 = m_v[W - 2, s]
        mt1 = m_v[W - 1, s]
        o_v[WD - 3, s] = TWO_THIRD * mt0 + THIRD * mt1
        o_v[WD - 2, s] = mt1
        o_v[WD - 1, s] = THIRD * mt1


def _body(x_hbm, out_hbm, a0_v, b0_v, a1_v, b1_v, o1_v, o2_v,
          in_sem0, in_sem1, out_sem1, out_sem2):
    wid = lax.axis_index("s") * NC + lax.axis_index("c")

    def unit_rows(t):
        g = wid * UPW + t
        bb = g // UNITS_PER_IMG
        p = g - bb * UNITS_PER_IMG
        is_edge = p == UNITS_PER_IMG - 1
        ra = jnp.where(is_edge, 0, jnp.minimum(p, H - 2))
        rb = jnp.where(is_edge, H - 1, ra + 1)
        return bb, p, is_edge, ra, rb

    def start_in(t, av, bv, sem):
        bb, _, _, ra, rb = unit_rows(t)
        pltpu.async_copy(x_hbm.at[bb, ra], av, sem)
        pltpu.async_copy(x_hbm.at[bb, rb], bv, sem)

    def wait_in(av, bv, sem):
        pltpu.make_async_copy(x_hbm.at[0, 0], av, sem).wait()
        pltpu.make_async_copy(x_hbm.at[0, 0], bv, sem).wait()

    def do_unit(t, av, bv, sem_cur, av_n, bv_n, sem_next):
        bb, p, is_edge, _, _ = unit_rows(t)
        u1 = jnp.where(is_edge, 0, 2 * p + 1)
        u2 = jnp.where(is_edge, HD - 1, 2 * p + 2)

        is_lo = p == 0
        is_hi = p == UNITS_PER_IMG - 2

        def wsel(lo, hi, edge, default):
            r = jnp.where(is_lo, _splat(lo), _splat(default))
            r = jnp.where(is_hi, _splat(hi), r)
            return jnp.where(is_edge, _splat(edge), r)

        a1 = wsel(1.0, TWO_THIRD, THIRD, 0.75)
        b1 = wsel(0.0, THIRD, 0.0, 0.25)
        a2 = wsel(THIRD, 0.0, 0.0, 0.25)
        b2 = wsel(TWO_THIRD, 1.0, THIRD, 0.75)

        wait_in(av, bv, sem_cur)

        @pl.when(t + 1 < UPW)
        def _():
            start_in(t + 1, av_n, bv_n, sem_next)

        @plsc.parallel_loop(0, W)
        def hrow(w):
            for cc in range(CV):
                s = pl.ds(cc * LANES, LANES)
                xa = av[w, s]
                xb = bv[w, s]
                av[w, s] = a1 * xa + b1 * xb
                bv[w, s] = a2 * xa + b2 * xb

        @pl.when(t > 0)
        def _():
            pltpu.make_async_copy(out_hbm.at[0, 0], o1_v, out_sem1).wait()

        _col_mix(av, o1_v)
        pltpu.async_copy(o1_v, out_hbm.at[bb, u1], out_sem1)

        @pl.when(t > 0)
        def _():
            pltpu.make_async_copy(out_hbm.at[0, 0], o2_v, out_sem2).wait()

        _col_mix(bv, o2_v)
        pltpu.async_copy(o2_v, out_hbm.at[bb, u2], out_sem2)

    start_in(0, a0_v, b0_v, in_sem0)

    def pair(q, carry):
        do_unit(2 * q, a0_v, b0_v, in_sem0, a1_v, b1_v, in_sem1)
        do_unit(2 * q + 1, a1_v, b1_v, in_sem1, a0_v, b0_v, in_sem0)
        return carry

    lax.fori_loop(0, UPW // 2, pair, 0)

    pltpu.make_async_copy(out_hbm.at[0, 0], o1_v, out_sem1).wait()
    pltpu.make_async_copy(out_hbm.at[0, 0], o2_v, out_sem2).wait()


@jax.jit
def _upsample(x):
    mesh = plsc.VectorSubcoreMesh(core_axis_name="c", subcore_axis_name="s")
    f = functools.partial(
        pl.kernel,
        mesh=mesh,
        out_type=jax.ShapeDtypeStruct((B, HD, WD, C), jnp.float32),
        scratch_types=[
            pltpu.VMEM((W, C), jnp.float32),
            pltpu.VMEM((W, C), jnp.float32),
            pltpu.VMEM((W, C), jnp.float32),
            pltpu.VMEM((W, C), jnp.float32),
            pltpu.VMEM((WD, C), jnp.float32),
            pltpu.VMEM((WD, C), jnp.float32),
            pltpu.SemaphoreType.DMA,
            pltpu.SemaphoreType.DMA,
            pltpu.SemaphoreType.DMA,
            pltpu.SemaphoreType.DMA,
        ],
    )(_body)
    return f(x)


def kernel(inputs):
    return _upsample(inputs)


# speedup vs baseline: 1.5894x; 1.0011x over previous
import functools

import jax
import jax.numpy as jnp
from jax import lax
from jax.experimental import pallas as pl
from jax.experimental.pallas import tpu as pltpu
from jax.experimental.pallas import tpu_sc as plsc

B, H, W, C = 4, 112, 112, 96
HD, WD = 2 * H, 2 * W
LANES = 16
CV = C // LANES
NC, NS = 2, 16
NW = NC * NS
UNITS_PER_IMG = H
UNITS = B * UNITS_PER_IMG
UPW = UNITS // NW

THIRD = 1.0 / 3.0
TWO_THIRD = 2.0 / 3.0


def _splat(val):
    return jnp.full((LANES,), val, jnp.float32)


def _col_mix(m_v, o_v):
    @plsc.parallel_loop(1, W - 2, unroll=2)
    def jbody(j):
        for cc in range(CV):
            s = pl.ds(cc * LANES, LANES)
            mj = m_v[j, s]
            mj1 = m_v[j + 1, s]
            o_v[2 * j + 1, s] = 0.75 * mj + 0.25 * mj1
            o_v[2 * j + 2, s] = 0.25 * mj + 0.75 * mj1

    for cc in range(CV):
        s = pl.ds(cc * LANES, LANES)
        m0 = m_v[0, s]
        m1 = m_v[1, s]
        o_v[0, s] = THIRD * m0
        o_v[1, s] = m0
        o_v[2, s] = THIRD * m0 + TWO_THIRD * m1
        mt0 = m_v[W - 2, s]
        mt1 = m_v[W - 1, s]
        o_v[WD - 3, s] = TWO_THIRD * mt0 + THIRD * mt1
        o_v[WD - 2, s] = mt1
        o_v[WD - 1, s] = THIRD * mt1


def _body(x_hbm, out_hbm, a0_v, b0_v, a1_v, b1_v, o1_v, o2_v,
          in_sem0, in_sem1, out_sem1, out_sem2):
    wid = lax.axis_index("s") * NC + lax.axis_index("c")

    def unit_rows(t):
        g = wid * UPW + t
        bb = g // UNITS_PER_IMG
        p = g - bb * UNITS_PER_IMG
        is_edge = p == UNITS_PER_IMG - 1
        ra = jnp.where(is_edge, 0, jnp.minimum(p, H - 2))
        rb = jnp.where(is_edge, H - 1, ra + 1)
        return bb, p, is_edge, ra, rb

    def start_in(t, av, bv, sem):
        bb, _, _, ra, rb = unit_rows(t)
        pltpu.async_copy(x_hbm.at[bb, ra], av, sem)
        pltpu.async_copy(x_hbm.at[bb, rb], bv, sem)

    def wait_in(av, bv, sem):
        pltpu.make_async_copy(x_hbm.at[0, 0], av, sem).wait()
        pltpu.make_async_copy(x_hbm.at[0, 0], bv, sem).wait()

    def do_unit(t, av, bv, sem_cur, av_n, bv_n, sem_next):
        bb, p, is_edge, _, _ = unit_rows(t)
        u1 = jnp.where(is_edge, 0, 2 * p + 1)
        u2 = jnp.where(is_edge, HD - 1, 2 * p + 2)

        is_lo = p == 0
        is_hi = p == UNITS_PER_IMG - 2

        def wsel(lo, hi, edge, default):
            r = jnp.where(is_lo, _splat(lo), _splat(default))
            r = jnp.where(is_hi, _splat(hi), r)
            return jnp.where(is_edge, _splat(edge), r)

        a1 = wsel(1.0, TWO_THIRD, THIRD, 0.75)
        b1 = wsel(0.0, THIRD, 0.0, 0.25)
        a2 = wsel(THIRD, 0.0, 0.0, 0.25)
        b2 = wsel(TWO_THIRD, 1.0, THIRD, 0.75)

        wait_in(av, bv, sem_cur)

        @pl.when(t + 1 < UPW)
        def _():
            start_in(t + 1, av_n, bv_n, sem_next)

        @plsc.parallel_loop(0, W, unroll=2)
        def hrow(w):
            for cc in range(CV):
                s = pl.ds(cc * LANES, LANES)
                xa = av[w, s]
                xb = bv[w, s]
                av[w, s] = a1 * xa + b1 * xb
                bv[w, s] = a2 * xa + b2 * xb

        @pl.when(t > 0)
        def _():
            pltpu.make_async_copy(out_hbm.at[0, 0], o1_v, out_sem1).wait()

        _col_mix(av, o1_v)
        pltpu.async_copy(o1_v, out_hbm.at[bb, u1], out_sem1)

        @pl.when(t > 0)
        def _():
            pltpu.make_async_copy(out_hbm.at[0, 0], o2_v, out_sem2).wait()

        _col_mix(bv, o2_v)
        pltpu.async_copy(o2_v, out_hbm.at[bb, u2], out_sem2)

    start_in(0, a0_v, b0_v, in_sem0)

    def pair(q, carry):
        do_unit(2 * q, a0_v, b0_v, in_sem0, a1_v, b1_v, in_sem1)
        do_unit(2 * q + 1, a1_v, b1_v, in_sem1, a0_v, b0_v, in_sem0)
        return carry

    lax.fori_loop(0, UPW // 2, pair, 0)

    pltpu.make_async_copy(out_hbm.at[0, 0], o1_v, out_sem1).wait()
    pltpu.make_async_copy(out_hbm.at[0, 0], o2_v, out_sem2).wait()


@jax.jit
def _upsample(x):
    mesh = plsc.VectorSubcoreMesh(core_axis_name="c", subcore_axis_name="s")
    f = functools.partial(
        pl.kernel,
        mesh=mesh,
        out_type=jax.ShapeDtypeStruct((B, HD, WD, C), jnp.float32),
        scratch_types=[
            pltpu.VMEM((W, C), jnp.float32),
            pltpu.VMEM((W, C), jnp.float32),
            pltpu.VMEM((W, C), jnp.float32),
            pltpu.VMEM((W, C), jnp.float32),
            pltpu.VMEM((WD, C), jnp.float32),
            pltpu.VMEM((WD, C), jnp.float32),
            pltpu.SemaphoreType.DMA,
            pltpu.SemaphoreType.DMA,
            pltpu.SemaphoreType.DMA,
            pltpu.SemaphoreType.DMA,
        ],
    )(_body)
    return f(x)


def kernel(inputs):
    return _upsample(inputs)


# speedup vs baseline: 1.6370x; 1.0300x over previous
import functools

import jax
import jax.numpy as jnp
from jax import lax
from jax.experimental import pallas as pl
from jax.experimental.pallas import tpu as pltpu
from jax.experimental.pallas import tpu_sc as plsc

B, H, W, C = 4, 112, 112, 96
HD, WD = 2 * H, 2 * W
LANES = 16
CV = C // LANES
NC, NS = 2, 16
NW = NC * NS
UNITS_PER_IMG = H
UNITS = B * UNITS_PER_IMG
UPW = UNITS // NW

THIRD = 1.0 / 3.0
TWO_THIRD = 2.0 / 3.0


def _splat(val):
    return jnp.full((LANES,), val, jnp.float32)


def _col_mix(m_v, o_v):
    @plsc.parallel_loop(1, W - 2, unroll=2)
    def jbody(j):
        for cc in range(CV):
            s = pl.ds(cc * LANES, LANES)
            mj = m_v[j, s]
            mj1 = m_v[j + 1, s]
            o_v[2 * j + 1, s] = 0.75 * mj + 0.25 * mj1
            o_v[2 * j + 2, s] = 0.25 * mj + 0.75 * mj1

    for cc in range(CV):
        s = pl.ds(cc * LANES, LANES)
        m0 = m_v[0, s]
        m1 = m_v[1, s]
        o_v[0, s] = THIRD * m0
        o_v[1, s] = m0
        o_v[2, s] = THIRD * m0 + TWO_THIRD * m1
        mt0 = m_v[W - 2, s]
        mt1 = m_v[W - 1, s]
        o_v[WD - 3, s] = TWO_THIRD * mt0 + THIRD * mt1
        o_v[WD - 2, s] = mt1
        o_v[WD - 1, s] = THIRD * mt1


def _body(x_hbm, out_hbm, r0_v, r1_v, r2_v, m1_v, m2_v, o1_v, o2_v,
          in_sem, out_sem1, out_sem2):
    wid = lax.axis_index("s") * NC + lax.axis_index("c")

    def unit_info(t):
        g = wid * UPW + t
        bb = g // UNITS_PER_IMG
        p = g - bb * UNITS_PER_IMG
        is_edge = p == UNITS_PER_IMG - 1
        rfresh = jnp.where(is_edge, 0, jnp.minimum(p, H - 2) + 1)
        return bb, p, is_edge, rfresh

    def start_fresh(t, rf):
        bb, _, _, rfresh = unit_info(t)
        pltpu.async_copy(x_hbm.at[bb, rfresh], rf, in_sem)

    def wait_one_row(rf):
        pltpu.make_async_copy(x_hbm.at[0, 0], rf, in_sem).wait()

    def do_unit(t, rc, rf, rn):
        bb, p, is_edge, _ = unit_info(t)
        u1 = jnp.where(is_edge, 0, 2 * p + 1)
        u2 = jnp.where(is_edge, HD - 1, 2 * p + 2)

        is_lo = p == 0
        is_hi = p == UNITS_PER_IMG - 2

        def wsel(lo, hi, edge, default):
            r = jnp.where(is_lo, _splat(lo), _splat(default))
            r = jnp.where(is_hi, _splat(hi), r)
            return jnp.where(is_edge, _splat(edge), r)

        wc1 = wsel(1.0, TWO_THIRD, 0.0, 0.75)
        wf1 = wsel(0.0, THIRD, THIRD, 0.25)
        wc2 = wsel(THIRD, 0.0, THIRD, 0.25)
        wf2 = wsel(TWO_THIRD, 1.0, 0.0, 0.75)

        wait_one_row(rf)

        @pl.when(t + 1 < UPW)
        def _():
            start_fresh(t + 1, rn)

        @plsc.parallel_loop(0, W, unroll=2)
        def hrow(w):
            for cc in range(CV):
                s = pl.ds(cc * LANES, LANES)
                xc = rc[w, s]
                xf = rf[w, s]
                m1_v[w, s] = wc1 * xc + wf1 * xf
                m2_v[w, s] = wc2 * xc + wf2 * xf

        @pl.when(t > 0)
        def _():
            pltpu.make_async_copy(out_hbm.at[0, 0], o1_v, out_sem1).wait()

        _col_mix(m1_v, o1_v)
        pltpu.async_copy(o1_v, out_hbm.at[bb, u1], out_sem1)

        @pl.when(t > 0)
        def _():
            pltpu.make_async_copy(out_hbm.at[0, 0], o2_v, out_sem2).wait()

        _col_mix(m2_v, o2_v)
        pltpu.async_copy(o2_v, out_hbm.at[bb, u2], out_sem2)

    g0 = wid * UPW
    bb0 = g0 // UNITS_PER_IMG
    p0 = g0 - bb0 * UNITS_PER_IMG
    pltpu.async_copy(x_hbm.at[bb0, p0], r0_v, in_sem)
    pltpu.make_async_copy(x_hbm.at[0, 0], r0_v, in_sem).wait()
    start_fresh(0, r1_v)

    bufs = (r0_v, r1_v, r2_v)

    def triple(q, carry):
        do_unit(3 * q, bufs[0], bufs[1], bufs[2])
        do_unit(3 * q + 1, bufs[1], bufs[2], bufs[0])
        do_unit(3 * q + 2, bufs[2], bufs[0], bufs[1])
        return carry

    lax.fori_loop(0, UPW // 3, triple, 0)
    do_unit(UPW - 2, bufs[0], bufs[1], bufs[2])
    do_unit(UPW - 1, bufs[1], bufs[2], bufs[0])

    pltpu.make_async_copy(out_hbm.at[0, 0], o1_v, out_sem1).wait()
    pltpu.make_async_copy(out_hbm.at[0, 0], o2_v, out_sem2).wait()


@jax.jit
def _upsample(x):
    mesh = plsc.VectorSubcoreMesh(core_axis_name="c", subcore_axis_name="s")
    f = functools.partial(
        pl.kernel,
        mesh=mesh,
        out_type=jax.ShapeDtypeStruct((B, HD, WD, C), jnp.float32),
        scratch_types=[
            pltpu.VMEM((W, C), jnp.float32),
            pltpu.VMEM((W, C), jnp.float32),
            pltpu.VMEM((W, C), jnp.float32),
            pltpu.VMEM((W, C), jnp.float32),
            pltpu.VMEM((W, C), jnp.float32),
            pltpu.VMEM((WD, C), jnp.float32),
            pltpu.VMEM((WD, C), jnp.float32),
            pltpu.SemaphoreType.DMA,
            pltpu.SemaphoreType.DMA,
            pltpu.SemaphoreType.DMA,
        ],
    )(_body)
    return f(x)


def kernel(inputs):
    return _upsample(inputs)
